# baseline (speedup 1.0000x reference)
import jax
import jax.numpy as jnp
from jax.experimental import pallas as pl
from jax.experimental.pallas import tpu as pltpu


def _trim_passthrough_kernel(x_ref, v_ref, m_ref, xo_ref, vo_ref, mo_ref,
                             sem_x, sem_v):
    cx = pltpu.make_async_copy(x_ref, xo_ref, sem_x)
    cv = pltpu.make_async_copy(v_ref, vo_ref, sem_v)
    cx.start()
    cv.start()
    mo_ref[...] = m_ref[...] != 0.0
    cx.wait()
    cv.wait()


def kernel(x, v, mask):
    out = pl.pallas_call(
        _trim_passthrough_kernel,
        in_specs=[
            pl.BlockSpec(memory_space=pl.ANY),
            pl.BlockSpec(memory_space=pl.ANY),
            pl.BlockSpec(mask.shape, lambda: (0, 0, 0)),
        ],
        out_specs=[
            pl.BlockSpec(memory_space=pl.ANY),
            pl.BlockSpec(memory_space=pl.ANY),
            pl.BlockSpec(mask.shape, lambda: (0, 0, 0)),
        ],
        out_shape=[
            jax.ShapeDtypeStruct(x.shape, x.dtype),
            jax.ShapeDtypeStruct(v.shape, v.dtype),
            jax.ShapeDtypeStruct(mask.shape, jnp.bool_),
        ],
        scratch_shapes=[pltpu.SemaphoreType.DMA, pltpu.SemaphoreType.DMA],
    )(x, v, mask)
    return (out[0], out[1], out[2])


# speedup vs baseline: 20.5806x; 20.5806x over previous
import jax
import jax.numpy as jnp
from jax.experimental import pallas as pl
from jax.experimental.pallas import tpu as pltpu

_N_BLOCKS = 16


def _trim_passthrough_kernel(x_ref, v_ref, m_ref, xo_ref, vo_ref, mo_ref):
    xo_ref[...] = x_ref[...]

    @pl.when(pl.program_id(0) == 0)
    def _():
        vo_ref[...] = v_ref[...]
        mo_ref[...] = m_ref[...] != 0.0


def kernel(x, v, mask):
    x2 = x.reshape(-1, x.shape[-1])
    v2 = v.reshape(-1, v.shape[-1])
    m2 = mask.reshape(-1, mask.shape[-1])
    rows = x2.shape[0] // _N_BLOCKS
    out = pl.pallas_call(
        _trim_passthrough_kernel,
        grid=(_N_BLOCKS,),
        in_specs=[
            pl.BlockSpec((rows, x2.shape[1]), lambda i: (i, 0)),
            pl.BlockSpec(v2.shape, lambda i: (0, 0)),
            pl.BlockSpec(m2.shape, lambda i: (0, 0)),
        ],
        out_specs=[
            pl.BlockSpec((rows, x2.shape[1]), lambda i: (i, 0)),
            pl.BlockSpec(v2.shape, lambda i: (0, 0)),
            pl.BlockSpec(m2.shape, lambda i: (0, 0)),
        ],
        out_shape=[
            jax.ShapeDtypeStruct(x2.shape, x.dtype),
            jax.ShapeDtypeStruct(v2.shape, v.dtype),
            jax.ShapeDtypeStruct(m2.shape, jnp.bool_),
        ],
    )(x2, v2, m2)
    return (out[0].reshape(x.shape), out[1].reshape(v.shape),
            out[2].reshape(mask.shape))


# speedup vs baseline: 26.4203x; 1.2838x over previous
import jax
import jax.numpy as jnp
from jax.experimental import pallas as pl
from jax.experimental.pallas import tpu as pltpu

_SPLIT = 1


def _trim_passthrough_kernel(x_ref, v_ref, m_ref, xo_ref, vo_ref, mo_ref):
    xo_ref[...] = x_ref[...]

    @pl.when(pl.program_id(0) == 0)
    def _():
        vo_ref[...] = v_ref[...]
        mo_ref[...] = m_ref[...] != 0.0


def kernel(x, v, mask):
    B, H, L = x.shape
    h = H // _SPLIT
    xspec = pl.BlockSpec((1, h, L), lambda i: (i // _SPLIT, i % _SPLIT, 0))
    out = pl.pallas_call(
        _trim_passthrough_kernel,
        grid=(B * _SPLIT,),
        in_specs=[
            xspec,
            pl.BlockSpec(v.shape, lambda i: (0, 0, 0)),
            pl.BlockSpec(mask.shape, lambda i: (0, 0, 0)),
        ],
        out_specs=[
            xspec,
            pl.BlockSpec(v.shape, lambda i: (0, 0, 0)),
            pl.BlockSpec(mask.shape, lambda i: (0, 0, 0)),
        ],
        out_shape=[
            jax.ShapeDtypeStruct(x.shape, x.dtype),
            jax.ShapeDtypeStruct(v.shape, v.dtype),
            jax.ShapeDtypeStruct(mask.shape, jnp.bool_),
        ],
    )(x, v, mask)
    return (out[0], out[1], out[2])
